# Initial kernel scaffold; baseline (speedup 1.0000x reference)
import jax
import jax.numpy as jnp
from jax.experimental import pallas as pl


def kernel(px, pair_i, pair_j, diff, i1, W_pi, W_pp):
    raise NotImplementedError("write your pallas kernel here")


# speedup vs baseline: 1.0161x; 1.0161x over previous
import jax
import jax.numpy as jnp
from jax.experimental import pallas as pl


def _mm_body(x_ref, w_ref, o_ref):
    o_ref[...] = jax.lax.dot_general(
        x_ref[...], w_ref[...],
        (((1,), (1,)), ((), ())),
        preferred_element_type=jnp.float32,
    )


def _mm_t(x2d, w):
    R, D = x2d.shape
    BR = 3000
    assert R % BR == 0
    return pl.pallas_call(
        _mm_body,
        grid=(R // BR,),
        in_specs=[
            pl.BlockSpec((BR, D), lambda i: (i, 0)),
            pl.BlockSpec((D, D), lambda i: (0, 0)),
        ],
        out_specs=pl.BlockSpec((BR, D), lambda i: (i, 0)),
        out_shape=jax.ShapeDtypeStruct((R, D), jnp.float32),
    )(x2d, w)


def kernel(px, pair_i, pair_j, diff, i1, W_pi, W_pp):
    N, _, D = px.shape
    y = _mm_t(px.reshape(N * 3, D), W_pi).reshape(N, 3, D)
    g = jnp.take(y, pair_i, axis=0) + jnp.take(y, pair_j, axis=0)
    ix = g * i1 + diff[..., None] * i1
    acc = jnp.zeros_like(px).at[pair_i].add(ix)
    px_new = _mm_t(acc.reshape(N * 3, D), W_pp).reshape(N, 3, D)
    return (px_new, ix)


# speedup vs baseline: 5.1340x; 5.0524x over previous
import functools

import jax
import jax.numpy as jnp
from jax import lax
from jax.experimental import pallas as pl
from jax.experimental.pallas import tpu as pltpu
from jax.experimental.pallas import tpu_sc as plsc

_N = 10000
_E = 160000
_D = 256
_D3 = 3 * _D

_NC, _NS, _L = 2, 16, 16
_NW = _NC * _NS
_EW = _E // _NW
_KA = 40
_NBA = _EW // _KA

_ZB = 5120
_ZT = _ZB // _NS
_AP = 2 * _ZB + 2 * 64
_KB = 64
_ESL = _E // _NS
_LSZ = _ESL + 2 * _KB

def _phase_a_body(y_hbm, pi_hbm, pj_hbm, dif_hbm, i1_hbm, ix_hbm,
                  idxi_v, idxj_v, yi_v, yj_v, i1_v, dif_v, out_v, sem):
    cid = lax.axis_index("c")
    sid = lax.axis_index("s")
    wid = sid * _NC + cid
    base0 = wid * _EW

    def block(b, carry):
        base = base0 + b * _KA
        pltpu.sync_copy(pi_hbm.at[pl.ds(base, _KA)], idxi_v)
        pltpu.sync_copy(pj_hbm.at[pl.ds(base, _KA)], idxj_v)
        pltpu.sync_copy(i1_hbm.at[pl.ds(base, _KA)], i1_v)
        pltpu.sync_copy(dif_hbm.at[pl.ds(base, _KA)], dif_v)
        pltpu.async_copy(y_hbm.at[idxi_v], yi_v, sem).wait()
        pltpu.async_copy(y_hbm.at[idxj_v], yj_v, sem).wait()

        def edge(k, c2):
            dbc = [dif_v[k, pl.ds(c3 * _L, _L)] for c3 in range(3)]
            for v in range(_D // _L):
                i1v = i1_v[k, pl.ds(v * _L, _L)]
                for c3 in range(3):
                    off = c3 * _D + v * _L
                    g = yi_v[k, pl.ds(off, _L)] + yj_v[k, pl.ds(off, _L)]
                    out_v[k, pl.ds(off, _L)] = (g + dbc[c3]) * i1v
            return c2

        lax.fori_loop(0, _KA, edge, 0)
        pltpu.sync_copy(out_v, ix_hbm.at[pl.ds(base, _KA)])
        return carry

    lax.fori_loop(0, _NBA, block, 0)


@functools.lru_cache(maxsize=None)
def _get_phase_a():
    mesh = plsc.VectorSubcoreMesh(
        core_axis_name="c", subcore_axis_name="s",
        num_cores=_NC, num_subcores=_NS)
    return pl.kernel(
        _phase_a_body,
        out_type=jax.ShapeDtypeStruct((_E, _D3), jnp.float32),
        mesh=mesh,
        scratch_types=[
            pltpu.VMEM((_KA,), jnp.int32),
            pltpu.VMEM((_KA,), jnp.int32),
            pltpu.VMEM((_KA, _D3), jnp.float32),
            pltpu.VMEM((_KA, _D3), jnp.float32),
            pltpu.VMEM((_KA, _D), jnp.float32),
            pltpu.VMEM((_KA, 3 * _L), jnp.float32),
            pltpu.VMEM((_KA, _D3), jnp.float32),
            pltpu.SemaphoreType.DMA,
        ],
    )


_BE = 640
_NBK = _E // _BE


def _scat_body(pi_ref, ix_ref, o_ref):
    @pl.when(pl.program_id(0) == 0)
    def _():
        o_ref[...] = jnp.zeros_like(o_ref)

    def edge8(g, c):
        base = pl.multiple_of(24 * g, 8)
        blk = ix_ref[pl.ds(base, 24), :]
        for j in range(8):
            idx = pi_ref[0, 0, 8 * g + j]
            off = pl.multiple_of(8 * idx, 8)
            o_ref[pl.ds(off, 3), :] = (
                o_ref[pl.ds(off, 3), :] + blk[3 * j:3 * j + 3, :])
        return c

    lax.fori_loop(0, _BE // 8, edge8, 0)


def _scatter_tc(ix2, pair_i):
    pi3 = pair_i.reshape(_NBK, 1, _BE)
    halves = []
    for h in range(2):
        halves.append(pl.pallas_call(
            _scat_body,
            grid=(_NBK,),
            in_specs=[
                pl.BlockSpec((1, 1, _BE), lambda i: (i, 0, 0),
                             memory_space=pltpu.SMEM),
                pl.BlockSpec((3 * _BE, _D // 2),
                             lambda i, h=h: (i, h)),
            ],
            out_specs=pl.BlockSpec((_N * 8, _D // 2), lambda i: (0, 0)),
            out_shape=jax.ShapeDtypeStruct((_N * 8, _D // 2), jnp.float32),
            compiler_params=pltpu.CompilerParams(
                vmem_limit_bytes=60 * 1024 * 1024),
        )(pi3, ix2))
    return halves


def _mm_body(x_ref, w_ref, o_ref):
    o_ref[...] = lax.dot_general(
        x_ref[...], w_ref[...], (((1,), (1,)), ((), ())),
        preferred_element_type=jnp.float32)


def _mm_t(x2d, w):
    R, D = x2d.shape
    BR = 3000
    return pl.pallas_call(
        _mm_body,
        grid=(R // BR,),
        in_specs=[
            pl.BlockSpec((BR, D), lambda i: (i, 0)),
            pl.BlockSpec((D, D), lambda i: (0, 0)),
        ],
        out_specs=pl.BlockSpec((BR, D), lambda i: (i, 0)),
        out_shape=jax.ShapeDtypeStruct((R, D), jnp.float32),
    )(x2d, w)


def kernel(px, pair_i, pair_j, diff, i1, W_pi, W_pp):
    y = _mm_t(px.reshape(_N * 3, _D), W_pi).reshape(_N, _D3)
    difp = jnp.broadcast_to(diff[:, :, None], (_E, 3, _L)).reshape(_E, 3 * _L)
    i1_2d = i1.reshape(_E, _D)
    ix = _get_phase_a()(y, pair_i, pair_j, difp, i1_2d)
    h0, h1 = _scatter_tc(ix.reshape(_E * 3, _D), pair_i)
    acc = jnp.concatenate(
        [h0.reshape(_N, 8, _D // 2)[:, :3, :],
         h1.reshape(_N, 8, _D // 2)[:, :3, :]], axis=2).reshape(_N * 3, _D)
    px_new = _mm_t(acc, W_pp).reshape(_N, 3, _D)
    return (px_new, ix.reshape(_E, 3, _D))
